# Initial kernel scaffold; baseline (speedup 1.0000x reference)
import jax
import jax.numpy as jnp
from jax.experimental import pallas as pl


def kernel(x, edge_index, batch, W1, b1, W2, b2, W3, b3, W4, b4):
    raise NotImplementedError("write your pallas kernel here")


# speedup vs baseline: 23.7020x; 23.7020x over previous
import jax
import jax.numpy as jnp
from jax import lax
from jax.experimental import pallas as pl
from jax.experimental.pallas import tpu as pltpu
from jax.experimental.pallas import tpu_sc as plsc

N = 10000
E = 320000
G = 128
FW = 128
NC = 2
NS = 16
NW = NC * NS
CH = 128
NCH = 80
EPW = NCH * CH
EPAD = NW * EPW
NPAD = 10240
RPT = NPAD // NS
ZR = 128


def _sc_mesh():
    return plsc.VectorSubcoreMesh(core_axis_name="c", subcore_axis_name="s",
                                  num_cores=NC, num_subcores=NS)


NCHH = NCH // 2


def _agg_body(y_hbm, src_hbm, dst_hbm, z_hbm, out_hbm,
              src_v, dst_v, rows_a, rows_b, acc, sem_a, sem_b):
    c = lax.axis_index("c")
    s = lax.axis_index("s")
    wid = s * NC + c

    pltpu.sync_copy(z_hbm, rows_a)
    for t in range(RPT // ZR):
        pltpu.sync_copy(rows_a, acc.at[pl.ds(s * RPT + t * ZR, ZR)])
    plsc.subcore_barrier()

    def fire(j, rows, sem):
        pltpu.async_copy(y_hbm.at[src_v.at[j]], rows, sem)

    def drain(j, rows, sem):
        pltpu.make_async_copy(y_hbm.at[src_v.at[j]], rows, sem).wait()

    def scat(j, rows):
        pltpu.sync_copy(rows, acc.at[dst_v.at[j]], add=True)

    def step(j2, carry):
        j = j2 * 2
        fire(j + 1, rows_b, sem_b)
        drain(j, rows_a, sem_a)
        scat(j, rows_a)

        @pl.when(j + 2 < NCHH)
        def _():
            fire(j + 2, rows_a, sem_a)

        drain(j + 1, rows_b, sem_b)
        scat(j + 1, rows_b)
        return carry

    for half in range(2):
        pltpu.sync_copy(src_hbm.at[wid, pl.ds(half * NCHH, NCHH)], src_v)
        pltpu.sync_copy(dst_hbm.at[wid, pl.ds(half * NCHH, NCHH)], dst_v)
        fire(0, rows_a, sem_a)
        lax.fori_loop(0, NCHH // 2, step, 0)

    plsc.subcore_barrier()
    for t in range(RPT // ZR):
        pltpu.sync_copy(acc.at[pl.ds(s * RPT + t * ZR, ZR)], rows_a)
        pltpu.sync_copy(rows_a, out_hbm.at[c, pl.ds(s * RPT + t * ZR, ZR)])


_agg = pl.kernel(
    _agg_body,
    out_type=jax.ShapeDtypeStruct((NC, NPAD, FW), jnp.float32),
    mesh=_sc_mesh(),
    compiler_params=pltpu.CompilerParams(use_tc_tiling_on_sc=False),
    scratch_types=[
        pltpu.VMEM((NCHH, CH), jnp.int32),
        pltpu.VMEM((NCHH, CH), jnp.int32),
        pltpu.VMEM((CH, FW), jnp.float32),
        pltpu.VMEM((CH, FW), jnp.float32),
        pltpu.VMEM_SHARED((NPAD, FW), jnp.float32),
        pltpu.SemaphoreType.DMA,
        pltpu.SemaphoreType.DMA,
    ],
)


def _deg_body(dst_hbm, ones_hbm, z_hbm, out_hbm, dst_v, ones_v, zb, acc):
    c = lax.axis_index("c")
    s = lax.axis_index("s")
    wid = s * NC + c

    pltpu.sync_copy(ones_hbm, ones_v)
    pltpu.sync_copy(z_hbm, zb)
    for t in range(RPT // ZR):
        pltpu.sync_copy(zb, acc.at[pl.ds(s * RPT + t * ZR, ZR)])
    pltpu.sync_copy(dst_hbm.at[wid], dst_v)
    plsc.subcore_barrier()

    def step(j, carry):
        pltpu.sync_copy(ones_v, acc.at[dst_v.at[j]], add=True)
        return carry

    lax.fori_loop(0, NCH, step, 0)
    plsc.subcore_barrier()
    for t in range(RPT // ZR):
        pltpu.sync_copy(acc.at[pl.ds(s * RPT + t * ZR, ZR)], zb)
        pltpu.sync_copy(zb, out_hbm.at[c, pl.ds(s * RPT + t * ZR, ZR)])


_deg = pl.kernel(
    _deg_body,
    out_type=jax.ShapeDtypeStruct((NC, NPAD, FW), jnp.float32),
    mesh=_sc_mesh(),
    compiler_params=pltpu.CompilerParams(use_tc_tiling_on_sc=False),
    scratch_types=[
        pltpu.VMEM((NCH, CH), jnp.int32),
        pltpu.VMEM((CH, FW), jnp.float32),
        pltpu.VMEM((ZR, FW), jnp.float32),
        pltpu.VMEM_SHARED((NPAD, FW), jnp.float32),
    ],
)


def _pad_cols(v, width=FW):
    n, f = v.shape
    if f == width:
        return v
    return jnp.concatenate([v, jnp.zeros((n, width - f), jnp.float32)], axis=1)


def _k1_body(x_ref, w_ref, degp_ref, y_ref, dinv_ref):
    deg = degp_ref[0, :N, 0:1] + degp_ref[1, :N, 0:1] + 1.0
    dinv = lax.rsqrt(deg)
    dinv_ref[...] = dinv
    y = dinv * jnp.dot(x_ref[...], w_ref[...],
                       preferred_element_type=jnp.float32)
    y_ref[...] = _pad_cols(y)


def _make_kmid(f_in, f_out):
    def body(a_ref, y_ref, dinv_ref, b_ref, w_ref, o_ref):
        dinv = dinv_ref[...]
        pre = dinv * (a_ref[0, :N, :f_in] + a_ref[1, :N, :f_in]
                      + y_ref[:, :f_in]) + b_ref[...]
        h = jnp.maximum(pre, 0.0)
        y = dinv * jnp.dot(h, w_ref[...], preferred_element_type=jnp.float32)
        o_ref[...] = _pad_cols(y)

    return pl.pallas_call(
        body, out_shape=jax.ShapeDtypeStruct((N, FW), jnp.float32))


def _k5_body(a_ref, y_ref, dinv_ref, b_ref, batch_ref, o_ref):
    dinv = dinv_ref[...]
    pre = dinv * (a_ref[0, :N, :2] + a_ref[1, :N, :2]
                  + y_ref[:, :2]) + b_ref[...]
    oh = (batch_ref[...] == lax.broadcasted_iota(jnp.int32, (1, G), 1))
    oh = oh.astype(jnp.float32)
    cdims = (((0,), (0,)), ((), ()))
    sums = lax.dot_general(oh, pre, cdims, preferred_element_type=jnp.float32)
    cnts = lax.dot_general(oh, jnp.ones((N, 1), jnp.float32), cdims,
                           preferred_element_type=jnp.float32)
    pooled = sums / jnp.maximum(cnts, 1.0)
    m = jnp.max(pooled, axis=1, keepdims=True)
    o_ref[...] = pooled - m - jnp.log(
        jnp.sum(jnp.exp(pooled - m), axis=1, keepdims=True))


_k1 = pl.pallas_call(
    _k1_body,
    out_shape=[jax.ShapeDtypeStruct((N, FW), jnp.float32),
               jax.ShapeDtypeStruct((N, 1), jnp.float32)],
)

_k5 = pl.pallas_call(
    _k5_body,
    out_shape=jax.ShapeDtypeStruct((G, 2), jnp.float32),
)


def kernel(x, edge_index, batch, W1, b1, W2, b2, W3, b3, W4, b4):
    npad = EPAD - E
    pad_src = (jnp.arange(npad, dtype=jnp.int32) * 37) % N
    pad_dst = N + (jnp.arange(npad, dtype=jnp.int32) % (NPAD - N))
    src = jnp.concatenate([edge_index[0], pad_src]).reshape(NW, NCH, CH)
    dst = jnp.concatenate([edge_index[1], pad_dst]).reshape(NW, NCH, CH)
    zeros = jnp.zeros((ZR, FW), jnp.float32)
    ones = jnp.ones((CH, FW), jnp.float32)

    degp = _deg(dst, ones, zeros)
    y1, dinv = _k1(x, W1, degp)
    a1 = _agg(y1, src, dst, zeros)
    y2 = _make_kmid(64, 64)(a1, y1, dinv, b1.reshape(1, -1), W2)
    a2 = _agg(y2, src, dst, zeros)
    y3 = _make_kmid(64, 32)(a2, y2, dinv, b2.reshape(1, -1), W3)
    a3 = _agg(y3, src, dst, zeros)
    y4 = _make_kmid(32, 2)(a3, y3, dinv, b3.reshape(1, -1), W4)
    a4 = _agg(y4, src, dst, zeros)
    return _k5(a4, y4, dinv, b4.reshape(1, -1), batch.reshape(-1, 1))
